# baseline (speedup 1.0000x reference)
import functools

import jax
import jax.numpy as jnp
from jax import lax
from jax.experimental import pallas as pl
from jax.experimental.pallas import tpu as pltpu
from jax.experimental.pallas import tpu_sc as plsc

VOCAB = 50257
SEQ = 2048
HID = 768
BATCH = 4

NUM_CORES = 2
NUM_SUBCORES = 16
NW = NUM_CORES * NUM_SUBCORES
S_PER_W = SEQ // NW
LANES = 16
VECS_PER_ROW = HID // LANES
C = 32
CHUNKS = BATCH * S_PER_W // C
NBUF = 3


def _build():
    mesh = plsc.VectorSubcoreMesh(core_axis_name="c", subcore_axis_name="s")

    @functools.partial(
        pl.kernel,
        mesh=mesh,
        out_type=jax.ShapeDtypeStruct((BATCH * SEQ, HID), jnp.float32),
        scratch_types=[
            pltpu.VMEM((BATCH, S_PER_W), jnp.int32),
            pltpu.VMEM((C, HID), jnp.float32),
            pltpu.VMEM((S_PER_W, HID), jnp.float32),
            pltpu.VMEM((C, HID), jnp.float32),
            pltpu.VMEM((C, HID), jnp.float32),
            pltpu.SemaphoreType.DMA,
            pltpu.SemaphoreType.DMA,
            pltpu.SemaphoreType.DMA,
            pltpu.SemaphoreType.DMA,
            pltpu.SemaphoreType.DMA,
            pltpu.SemaphoreType.DMA,
            pltpu.SemaphoreType.DMA,
            pltpu.SemaphoreType.DMA,
        ],
    )
    def embed(ids_hbm, table_hbm, pos_hbm, out_hbm,
              idx_v, buf0, pos_v, buf1, buf2, isem, psem,
              g0, g1, g2, o0, o1, o2):
        wid = lax.axis_index("s") * NUM_CORES + lax.axis_index("c")
        s_base = wid * S_PER_W
        bufs = (buf0, buf1, buf2)
        gsems = (g0, g1, g2)
        osems = (o0, o1, o2)

        id_copies = [
            pltpu.async_copy(
                ids_hbm.at[b, pl.ds(s_base, S_PER_W)], idx_v.at[b], isem)
            for b in range(BATCH)
        ]
        pos_copy = pltpu.async_copy(
            pos_hbm.at[pl.ds(s_base, S_PER_W)], pos_v, psem)
        for cp in id_copies:
            cp.wait()

        def start_gather(k):
            b, half = divmod(k, 2)
            idx = idx_v.at[b, pl.ds(half * C, C)]
            return pltpu.async_copy(
                table_hbm.at[idx], bufs[k % NBUF], gsems[k % NBUF])

        gathers = [None] * CHUNKS
        outs = [None] * CHUNKS
        gathers[0] = start_gather(0)
        gathers[1] = start_gather(1)
        pos_copy.wait()

        for k in range(CHUNKS):
            b, half = divmod(k, 2)
            gathers[k].wait()
            buf = bufs[k % NBUF]
            pbase = half * C

            def _add(r, carry, _buf=buf, _pbase=pbase):
                for cc in range(VECS_PER_ROW):
                    sl = pl.ds(cc * LANES, LANES)
                    plsc.addupdate(_buf.at[r, sl], pos_v[_pbase + r, sl])
                return carry

            lax.fori_loop(0, C, _add, 0)

            flat = b * SEQ + s_base + half * C
            outs[k] = pltpu.async_copy(
                buf, out_hbm.at[pl.ds(flat, C)], osems[k % NBUF])
            if k + 2 < CHUNKS:
                if k >= 1:
                    outs[k - 1].wait()
                gathers[k + 2] = start_gather(k + 2)

        outs[CHUNKS - 3].wait()
        outs[CHUNKS - 2].wait()
        outs[CHUNKS - 1].wait()

    return embed


_embed = _build()


def kernel(input_ids, token_embeddings, position_embeddings):
    ids = input_ids.astype(jnp.int32)
    out = _embed(ids, token_embeddings, position_embeddings)
    return out.reshape(BATCH, SEQ, HID)


# speedup vs baseline: 1.1942x; 1.1942x over previous
import functools

import jax
import jax.numpy as jnp
from jax import lax
from jax.experimental import pallas as pl
from jax.experimental.pallas import tpu as pltpu
from jax.experimental.pallas import tpu_sc as plsc

VOCAB = 50257
SEQ = 2048
HID = 768
BATCH = 4

NUM_CORES = 2
NUM_SUBCORES = 16
NW = NUM_CORES * NUM_SUBCORES
S_PER_W = SEQ // NW
LANES = 16
VECS_PER_ROW = HID // LANES
G = 16
GROUPS = S_PER_W // G
NSET = 2


def _build():
    mesh = plsc.VectorSubcoreMesh(core_axis_name="c", subcore_axis_name="s")

    bufspec = pltpu.VMEM((G, HID), jnp.float32)

    @functools.partial(
        pl.kernel,
        mesh=mesh,
        out_type=jax.ShapeDtypeStruct((BATCH * SEQ, HID), jnp.float32),
        scratch_types=[
            pltpu.VMEM((BATCH, S_PER_W), jnp.int32),
            bufspec, bufspec, bufspec, bufspec, bufspec,
            bufspec, bufspec, bufspec, bufspec, bufspec,
            pltpu.SemaphoreType.DMA,
            pltpu.SemaphoreType.DMA,
            pltpu.SemaphoreType.DMA,
            pltpu.SemaphoreType.DMA,
            pltpu.SemaphoreType.DMA,
        ],
    )
    def embed(ids_hbm, table_hbm, pos_hbm, out_hbm,
              idx_v,
              p0, r00, r01, r02, r03,
              p1, r10, r11, r12, r13,
              isem, g0sem, g1sem, o0sem, o1sem):
        wid = lax.axis_index("s") * NUM_CORES + lax.axis_index("c")
        s_base = wid * S_PER_W
        posb = (p0, p1)
        rowb = ((r00, r01, r02, r03), (r10, r11, r12, r13))
        gsems = (g0sem, g1sem)
        osems = (o0sem, o1sem)

        id_copies = [
            pltpu.async_copy(
                ids_hbm.at[b, pl.ds(s_base, S_PER_W)], idx_v.at[b], isem)
            for b in range(BATCH)
        ]
        for cp in id_copies:
            cp.wait()

        def start_group(g):
            st = g % NSET
            cps = [pltpu.async_copy(
                pos_hbm.at[pl.ds(s_base + g * G, G)], posb[st], gsems[st])]
            for b in range(BATCH):
                idx = idx_v.at[b, pl.ds(g * G, G)]
                cps.append(pltpu.async_copy(
                    table_hbm.at[idx], rowb[st][b], gsems[st]))
            return cps

        def start_outs(g):
            st = g % NSET
            return [
                pltpu.async_copy(
                    rowb[st][b],
                    out_hbm.at[pl.ds(b * SEQ + s_base + g * G, G)],
                    osems[st])
                for b in range(BATCH)
            ]

        gathers = [None] * GROUPS
        outs = [None] * GROUPS
        gathers[0] = start_group(0)
        gathers[1] = start_group(1)

        for g in range(GROUPS):
            st = g % NSET
            for cp in gathers[g]:
                cp.wait()
            bufs = rowb[st]
            pbuf = posb[st]

            def _add(r, carry, _bufs=bufs, _pbuf=pbuf):
                for cc in range(VECS_PER_ROW):
                    sl = pl.ds(cc * LANES, LANES)
                    pv = _pbuf[r, sl]
                    for b in range(BATCH):
                        plsc.addupdate(_bufs[b].at[r, sl], pv)
                return carry

            lax.fori_loop(0, G, _add, 0)

            outs[g] = start_outs(g)
            if g + 2 < GROUPS:
                for cp in outs[g]:
                    cp.wait()
                gathers[g + 2] = start_group(g + 2)

        for g in (GROUPS - 2, GROUPS - 1):
            for cp in outs[g]:
                cp.wait()

    return embed


_embed = _build()


def kernel(input_ids, token_embeddings, position_embeddings):
    ids = input_ids.astype(jnp.int32)
    out = _embed(ids, token_embeddings, position_embeddings)
    return out.reshape(BATCH, SEQ, HID)


# speedup vs baseline: 1.1991x; 1.0041x over previous
import functools

import jax
import jax.numpy as jnp
from jax import lax
from jax.experimental import pallas as pl
from jax.experimental.pallas import tpu as pltpu
from jax.experimental.pallas import tpu_sc as plsc

VOCAB = 50257
SEQ = 2048
HID = 768
BATCH = 4

NUM_CORES = 2
NUM_SUBCORES = 16
NW = NUM_CORES * NUM_SUBCORES
S_PER_W = SEQ // NW
LANES = 16
VECS_PER_ROW = HID // LANES
G = 16
GROUPS = S_PER_W // G
NSET = 2


def _build():
    mesh = plsc.VectorSubcoreMesh(core_axis_name="c", subcore_axis_name="s")

    bufspec = pltpu.VMEM((G, HID), jnp.float32)

    @functools.partial(
        pl.kernel,
        mesh=mesh,
        out_type=jax.ShapeDtypeStruct((BATCH * SEQ, HID), jnp.float32),
        scratch_types=[
            pltpu.VMEM((BATCH, S_PER_W), jnp.int32),
            bufspec, bufspec, bufspec, bufspec, bufspec,
            bufspec, bufspec, bufspec, bufspec, bufspec,
            pltpu.SemaphoreType.DMA,
            pltpu.SemaphoreType.DMA,
            pltpu.SemaphoreType.DMA,
            pltpu.SemaphoreType.DMA,
            pltpu.SemaphoreType.DMA,
        ],
    )
    def embed(ids_hbm, table_hbm, pos_hbm, out_hbm,
              idx_v,
              p0, r00, r01, r02, r03,
              p1, r10, r11, r12, r13,
              isem, g0sem, g1sem, o0sem, o1sem):
        wid = lax.axis_index("s") * NUM_CORES + lax.axis_index("c")
        s_base = wid * S_PER_W
        posb = (p0, p1)
        rowb = ((r00, r01, r02, r03), (r10, r11, r12, r13))
        gsems = (g0sem, g1sem)
        osems = (o0sem, o1sem)

        def start_pos(g):
            st = g % NSET
            return pltpu.async_copy(
                pos_hbm.at[pl.ds(s_base + g * G, G)], posb[st], gsems[st])

        def start_gathers(g, pos_cp):
            st = g % NSET
            cps = [pos_cp]
            for b in range(BATCH):
                idx = idx_v.at[b, pl.ds(g * G, G)]
                cps.append(pltpu.async_copy(
                    table_hbm.at[idx], rowb[st][b], gsems[st]))
            return cps

        def start_group(g):
            return start_gathers(g, start_pos(g))

        pos0 = start_pos(0)
        pos1 = start_pos(1)
        id_copies = [
            pltpu.async_copy(
                ids_hbm.at[b, pl.ds(s_base, S_PER_W)], idx_v.at[b], isem)
            for b in range(BATCH)
        ]
        for cp in id_copies:
            cp.wait()

        def start_outs(g):
            st = g % NSET
            return [
                pltpu.async_copy(
                    rowb[st][b],
                    out_hbm.at[pl.ds(b * SEQ + s_base + g * G, G)],
                    osems[st])
                for b in range(BATCH)
            ]

        gathers = [None] * GROUPS
        outs = [None] * GROUPS
        gathers[0] = start_gathers(0, pos0)
        gathers[1] = start_gathers(1, pos1)

        for g in range(GROUPS):
            st = g % NSET
            for cp in gathers[g]:
                cp.wait()
            bufs = rowb[st]
            pbuf = posb[st]

            def _add(r, carry, _bufs=bufs, _pbuf=pbuf):
                for cc in range(VECS_PER_ROW):
                    sl = pl.ds(cc * LANES, LANES)
                    pv = _pbuf[r, sl]
                    for b in range(BATCH):
                        plsc.addupdate(_bufs[b].at[r, sl], pv)
                return carry

            lax.fori_loop(0, G, _add, 0)

            outs[g] = start_outs(g)
            if g + 2 < GROUPS:
                for cp in outs[g]:
                    cp.wait()
                gathers[g + 2] = start_group(g + 2)

        for g in (GROUPS - 2, GROUPS - 1):
            for cp in outs[g]:
                cp.wait()

    return embed


_embed = _build()


def kernel(input_ids, token_embeddings, position_embeddings):
    ids = input_ids.astype(jnp.int32)
    out = _embed(ids, token_embeddings, position_embeddings)
    return out.reshape(BATCH, SEQ, HID)


# speedup vs baseline: 1.2154x; 1.0136x over previous
import functools

import jax
import jax.numpy as jnp
from jax import lax
from jax.experimental import pallas as pl
from jax.experimental.pallas import tpu as pltpu
from jax.experimental.pallas import tpu_sc as plsc

VOCAB = 50257
SEQ = 2048
HID = 768
BATCH = 4

NUM_CORES = 2
NUM_SUBCORES = 16
NW = NUM_CORES * NUM_SUBCORES
S_PER_W = SEQ // NW
LANES = 16
VECS_PER_ROW = HID // LANES
G = 16
GROUPS = S_PER_W // G
NSET = 2


def _build():
    mesh = plsc.VectorSubcoreMesh(core_axis_name="c", subcore_axis_name="s")

    bufspec = pltpu.VMEM((G, HID), jnp.float32)

    @functools.partial(
        pl.kernel,
        mesh=mesh,
        out_type=jax.ShapeDtypeStruct((BATCH * SEQ, HID), jnp.float32),
        scratch_types=[
            pltpu.VMEM((BATCH, S_PER_W), jnp.int32),
            bufspec, bufspec, bufspec, bufspec, bufspec,
            bufspec, bufspec, bufspec, bufspec, bufspec,
            pltpu.SemaphoreType.DMA,
            pltpu.SemaphoreType.DMA,
            pltpu.SemaphoreType.DMA,
            pltpu.SemaphoreType.DMA,
            pltpu.SemaphoreType.DMA,
        ],
    )
    def embed(ids_hbm, table_hbm, pos_hbm, out_hbm,
              idx_v,
              p0, r00, r01, r02, r03,
              p1, r10, r11, r12, r13,
              isem, g0sem, g1sem, o0sem, o1sem):
        wid = lax.axis_index("s") * NUM_CORES + lax.axis_index("c")
        s_base = wid * S_PER_W
        posb = (p0, p1)
        rowb = ((r00, r01, r02, r03), (r10, r11, r12, r13))
        gsems = (g0sem, g1sem)
        osems = (o0sem, o1sem)

        def start_pos(g):
            st = g % NSET
            return pltpu.async_copy(
                pos_hbm.at[pl.ds(s_base + g * G, G)], posb[st], gsems[st])

        def start_gathers(g, pos_cp):
            st = g % NSET
            cps = [pos_cp]
            for b in range(BATCH):
                idx = idx_v.at[b, pl.ds(g * G, G)]
                cps.append(pltpu.async_copy(
                    table_hbm.at[idx], rowb[st][b], gsems[st]))
            return cps

        def start_group(g):
            return start_gathers(g, start_pos(g))

        pos0 = start_pos(0)
        pos1 = start_pos(1)
        id_copies = [
            pltpu.async_copy(
                ids_hbm.at[b, pl.ds(s_base, S_PER_W)], idx_v.at[b], isem)
            for b in range(BATCH)
        ]
        for cp in id_copies:
            cp.wait()

        def start_outs(g):
            st = g % NSET
            return [
                pltpu.async_copy(
                    rowb[st][b],
                    out_hbm.at[pl.ds(b * SEQ + s_base + g * G, G)],
                    osems[st])
                for b in range(BATCH)
            ]

        gathers = [None] * GROUPS
        outs = [None] * GROUPS
        gathers[0] = start_gathers(0, pos0)
        gathers[1] = start_gathers(1, pos1)

        for g in range(GROUPS):
            st = g % NSET
            for cp in gathers[g]:
                cp.wait()
            bufs = rowb[st]
            pbuf = posb[st]

            def _add(r, carry, _bufs=bufs, _pbuf=pbuf):
                for cc in range(VECS_PER_ROW):
                    sl = pl.ds(cc * LANES, LANES)
                    pv = _pbuf[r, sl]
                    for b in range(BATCH):
                        plsc.addupdate(_bufs[b].at[r, sl], pv)
                return carry

            lax.fori_loop(0, G, _add, 0)

            outs[g] = start_outs(g)
            if g + 2 < GROUPS:
                gn = g + 2
                stn = gn % NSET
                cps = [start_pos(gn)]
                for b in range(BATCH):
                    outs[g][b].wait()
                    idx = idx_v.at[b, pl.ds(gn * G, G)]
                    cps.append(pltpu.async_copy(
                        table_hbm.at[idx], rowb[stn][b], gsems[stn]))
                gathers[gn] = cps

        for g in (GROUPS - 2, GROUPS - 1):
            for cp in outs[g]:
                cp.wait()

    return embed


_embed = _build()


def kernel(input_ids, token_embeddings, position_embeddings):
    ids = input_ids.astype(jnp.int32)
    out = _embed(ids, token_embeddings, position_embeddings)
    return out.reshape(BATCH, SEQ, HID)
